# baseline (speedup 1.0000x reference)
import jax
import jax.numpy as jnp
import numpy as np
from jax.experimental import pallas as pl
from jax.experimental.pallas import tpu as pltpu

N_ALL = 20000
PRE_N = 4000
POST_N = 1000
THRESH = 0.7
NPAD = 4096
B = 128
NB = NPAD // B
NROWS = N_ALL + (NPAD - PRE_N)


def _decode7(parts_a, parts_d):
    xa, ya, za, la, wa, ha, ra = parts_a
    dx, dy, dz, dl, dw, dh, dr = parts_d
    diag = jnp.sqrt(la * la + wa * wa)
    xg = dx * diag + xa
    yg = dy * diag + ya
    zg = dz * ha + za
    lg = jnp.exp(dl) * la
    wg = jnp.exp(dw) * wa
    hg = jnp.exp(dh) * ha
    rg = dr + ra
    return xg, yg, zg, lg, wg, hg, rg


def _iou(rows, cols):
    rx1, rx2, ry1, ry2, rar = rows
    cx1, cx2, cy1, cy2, car = cols
    ix1 = jnp.maximum(rx1, cx1)
    ix2 = jnp.minimum(rx2, cx2)
    iy1 = jnp.maximum(ry1, cy1)
    iy2 = jnp.minimum(ry2, cy2)
    inter = jnp.clip(ix2 - ix1, 0.0) * jnp.clip(iy2 - iy1, 0.0)
    union = rar + car - inter
    return inter / jnp.maximum(union, 1e-8)


def _nms_body(idx_ref, packed_ref, scoT, prop_ref, masked_ref,
              gat_ref, geomT, geomW, a_ref, p_ref):
    f32 = jnp.float32

    def g_body(i, carry):
        j = idx_ref[i]
        gat_ref[pl.ds(i, 1), :] = packed_ref[pl.ds(j, 1), :]
        return carry

    jax.lax.fori_loop(0, NPAD, g_body, 0)

    g = gat_ref[...]
    pa = [g[:, i:i + 1] for i in range(7)]
    pd = [g[:, 8 + i:8 + i + 1] for i in range(7)]
    xgT, ygT, zgT, lgT, wgT, hgT, rgT = _decode7(pa, pd)
    zcol = jnp.zeros_like(xgT)
    prop_ref[...] = jnp.concatenate([xgT, ygT, zgT, lgT, wgT, hgT, rgT, zcol],
                                    axis=1)
    x1T = xgT - lgT * 0.5
    x2T = xgT + lgT * 0.5
    y1T = ygT - wgT * 0.5
    y2T = ygT + wgT * 0.5
    areaT = lgT * wgT
    onecol = jnp.ones_like(xgT)
    geomT[...] = jnp.concatenate(
        [x1T, x2T, y1T, y2T, areaT, onecol, zcol, zcol], axis=1)

    ri = jax.lax.broadcasted_iota(jnp.int32, (B, B), 0)
    ci = jax.lax.broadcasted_iota(jnp.int32, (B, B), 1)
    eyem = (ri == ci).astype(f32)

    for b in range(NB):
        c0 = b * B
        gb = geomT[c0:c0 + B, :]
        geomW[:, c0:c0 + B] = jnp.transpose(gb)

    for b in range(NB):
        c0 = b * B
        cols = [geomW[k:k + 1, c0:c0 + B] for k in range(5)]

        def a_body(a, sup):
            r0 = a * B
            gblk = geomT[pl.ds(r0, B), :]
            rows = [gblk[:, k:k + 1] for k in range(5)]
            keep_a = gblk[:, 5:6]
            iou = _iou(rows, cols)
            hit = jnp.where((iou > THRESH) & (keep_a > 0.0), 1.0, 0.0)
            return jnp.maximum(sup, jnp.max(hit, axis=0, keepdims=True))

        sup0 = geomW[6:7, c0:c0 + B]
        if b > 0:
            sup_prev = jax.lax.fori_loop(0, b, a_body, sup0)
        else:
            sup_prev = sup0

        rows_b = [geomT[c0:c0 + B, k:k + 1] for k in range(5)]
        iou_bb = _iou(rows_b, cols)
        M = jnp.where((iou_bb > THRESH) & (ci > ri), 1.0, 0.0)

        kl0 = 1.0 - sup_prev
        a_ref[...] = kl0
        p_ref[...] = kl0 - 1.0

        def w_cond(t):
            return jnp.logical_and(t < B,
                                   jnp.any(a_ref[...] != p_ref[...]))

        def w_body(t):
            a = a_ref[...]
            sup = jax.lax.dot_general(a, M, (((1,), (0,)), ((), ())),
                                      preferred_element_type=f32)
            p_ref[...] = a
            a_ref[...] = jnp.where(sup > 0.0, 0.0, kl0)
            return t + 1

        jax.lax.while_loop(w_cond, w_body, jnp.int32(0))
        keep_b = a_ref[...]

        keep_tall = jax.lax.dot_general(
            eyem, keep_b, (((1,), (1,)), ((), ())),
            preferred_element_type=f32)
        geomT[c0:c0 + B, 5:6] = keep_tall

    masked_ref[...] = jnp.where(geomT[:, 5:6] > 0.0, scoT[...], -1.0)


def _sel_body(idx_ref, prop_ref, out_ref):
    def g_body(i, carry):
        j = idx_ref[i]
        out_ref[pl.ds(i, 1), :] = prop_ref[pl.ds(j, 1), :]
        return carry

    jax.lax.fori_loop(0, out_ref.shape[0], g_body, 0)


def kernel(anchors_bbox3d, objectness, box_regression):
    f32 = jnp.float32
    scores = jax.nn.sigmoid(objectness)
    top_scores, top_idx = jax.lax.top_k(scores, PRE_N)

    pad = NPAD - PRE_N
    pad_anc = jnp.tile(
        jnp.array([[1.0e4, 1.0e4, 0.0, 1.0, 1.0, 1.0, 0.0]], f32), (pad, 1))
    anc_all = jnp.concatenate([anchors_bbox3d, pad_anc], axis=0)
    dl_all = jnp.concatenate([box_regression, jnp.zeros((pad, 7), f32)],
                             axis=0)
    packed = jnp.concatenate(
        [anc_all, jnp.zeros((NROWS, 1), f32),
         dl_all, jnp.zeros((NROWS, 1), f32)], axis=1)

    idxT = jnp.concatenate(
        [top_idx, N_ALL + jnp.arange(pad, dtype=top_idx.dtype)]
    ).astype(jnp.int32)
    scoT = jnp.concatenate(
        [top_scores, jnp.full((pad,), -2.0, f32)]).reshape(NPAD, 1)

    prop, masked = pl.pallas_call(
        _nms_body,
        in_specs=[
            pl.BlockSpec(memory_space=pltpu.SMEM),
            pl.BlockSpec(memory_space=pltpu.VMEM),
            pl.BlockSpec(memory_space=pltpu.VMEM),
        ],
        out_shape=[
            jax.ShapeDtypeStruct((NPAD, 8), f32),
            jax.ShapeDtypeStruct((NPAD, 1), f32),
        ],
        scratch_shapes=[
            pltpu.VMEM((NPAD, 16), f32),
            pltpu.VMEM((NPAD, 8), f32),
            pltpu.VMEM((8, NPAD), f32),
            pltpu.VMEM((1, B), f32),
            pltpu.VMEM((1, B), f32),
        ],
    )(idxT, packed, scoT)

    sel_scores, sel_idx = jax.lax.top_k(masked[:, 0], POST_N)
    selT = jnp.pad(sel_idx.astype(jnp.int32), (0, 24))
    out_g = pl.pallas_call(
        _sel_body,
        in_specs=[
            pl.BlockSpec(memory_space=pltpu.SMEM),
            pl.BlockSpec(memory_space=pltpu.VMEM),
        ],
        out_shape=jax.ShapeDtypeStruct((POST_N + 24, 8), f32),
    )(selT, prop)
    return jnp.concatenate(
        [out_g[:POST_N, :7], sel_scores[:, None]], axis=1)


# speedup vs baseline: 1.1627x; 1.1627x over previous
import jax
import jax.numpy as jnp
import numpy as np
from jax.experimental import pallas as pl
from jax.experimental.pallas import tpu as pltpu

N_ALL = 20000
PRE_N = 4000
POST_N = 1000
THRESH = 0.7
NPAD = 4096
B = 128
NB = NPAD // B
NROWS = N_ALL + (NPAD - PRE_N)


def _decode7(parts_a, parts_d):
    xa, ya, za, la, wa, ha, ra = parts_a
    dx, dy, dz, dl, dw, dh, dr = parts_d
    diag = jnp.sqrt(la * la + wa * wa)
    xg = dx * diag + xa
    yg = dy * diag + ya
    zg = dz * ha + za
    lg = jnp.exp(dl) * la
    wg = jnp.exp(dw) * wa
    hg = jnp.exp(dh) * ha
    rg = dr + ra
    return xg, yg, zg, lg, wg, hg, rg


def _iou(rows, cols):
    rx1, rx2, ry1, ry2, rar = rows
    cx1, cx2, cy1, cy2, car = cols
    ix1 = jnp.maximum(rx1, cx1)
    ix2 = jnp.minimum(rx2, cx2)
    iy1 = jnp.maximum(ry1, cy1)
    iy2 = jnp.minimum(ry2, cy2)
    inter = jnp.clip(ix2 - ix1, 0.0) * jnp.clip(iy2 - iy1, 0.0)
    union = rar + car - inter
    return inter / jnp.maximum(union, 1e-8)


def _nms_body(idx_ref, packed_ref, scoT, prop_ref, masked_ref,
              gat_ref, geomT, geomW, a_ref, p_ref):
    f32 = jnp.float32

    def g_body(i, carry):
        j = idx_ref[i]
        gat_ref[pl.ds(i, 1), :] = packed_ref[pl.ds(j, 1), :]
        return carry

    jax.lax.fori_loop(0, NPAD, g_body, 0)

    g = gat_ref[...]
    pa = [g[:, i:i + 1] for i in range(7)]
    pd = [g[:, 8 + i:8 + i + 1] for i in range(7)]
    xgT, ygT, zgT, lgT, wgT, hgT, rgT = _decode7(pa, pd)
    zcol = jnp.zeros_like(xgT)
    prop_ref[...] = jnp.concatenate([xgT, ygT, zgT, lgT, wgT, hgT, rgT, zcol],
                                    axis=1)
    x1T = xgT - lgT * 0.5
    x2T = xgT + lgT * 0.5
    y1T = ygT - wgT * 0.5
    y2T = ygT + wgT * 0.5
    areaT = lgT * wgT
    onecol = jnp.ones_like(xgT)
    geomT[...] = jnp.concatenate(
        [x1T, x2T, y1T, y2T, areaT, onecol, zcol, zcol], axis=1)

    ri = jax.lax.broadcasted_iota(jnp.int32, (B, B), 0)
    ci = jax.lax.broadcasted_iota(jnp.int32, (B, B), 1)
    eyem = (ri == ci).astype(f32)

    for b in range(NB):
        c0 = b * B
        gb = geomT[c0:c0 + B, :]
        geomW[:, c0:c0 + B] = jnp.transpose(gb)

    for b in range(NB):
        c0 = b * B
        cols = [geomW[k:k + 1, c0:c0 + B] for k in range(5)]

        def _a_step(r0, sup):
            gblk = geomT[pl.ds(r0, B), :]
            rows = [gblk[:, k:k + 1] for k in range(5)]
            keep_a = gblk[:, 5:6]
            iou = _iou(rows, cols)
            hit = jnp.where((iou > THRESH) & (keep_a > 0.0), 1.0, 0.0)
            return jnp.maximum(sup, jnp.max(hit, axis=0, keepdims=True))

        def a_body4(a4, sup):
            r0 = a4 * (4 * B)
            for u in range(4):
                sup = _a_step(r0 + u * B, sup)
            return sup

        sup_prev = geomW[6:7, c0:c0 + B]
        if b >= 4:
            sup_prev = jax.lax.fori_loop(0, b // 4, a_body4, sup_prev)
        for a in range(4 * (b // 4), b):
            sup_prev = _a_step(a * B, sup_prev)

        rows_b = [geomT[c0:c0 + B, k:k + 1] for k in range(5)]
        iou_bb = _iou(rows_b, cols)
        M = jnp.where((iou_bb > THRESH) & (ci > ri), 1.0, 0.0)

        kl0 = 1.0 - sup_prev
        a_ref[...] = kl0
        p_ref[...] = kl0 - 1.0

        def w_cond(t):
            return jnp.logical_and(t < B,
                                   jnp.any(a_ref[...] != p_ref[...]))

        def w_body(t):
            a = a_ref[...]
            sup = jax.lax.dot_general(a, M, (((1,), (0,)), ((), ())),
                                      preferred_element_type=f32)
            p_ref[...] = a
            a_ref[...] = jnp.where(sup > 0.0, 0.0, kl0)
            return t + 1

        jax.lax.while_loop(w_cond, w_body, jnp.int32(0))
        keep_b = a_ref[...]

        keep_tall = jax.lax.dot_general(
            eyem, keep_b, (((1,), (1,)), ((), ())),
            preferred_element_type=f32)
        geomT[c0:c0 + B, 5:6] = keep_tall

    masked_ref[...] = jnp.where(geomT[:, 5:6] > 0.0, scoT[...], -1.0)


def _sel_body(idx_ref, prop_ref, out_ref):
    def g_body(i, carry):
        j = idx_ref[i]
        out_ref[pl.ds(i, 1), :] = prop_ref[pl.ds(j, 1), :]
        return carry

    jax.lax.fori_loop(0, out_ref.shape[0], g_body, 0)


def kernel(anchors_bbox3d, objectness, box_regression):
    f32 = jnp.float32
    scores = jax.nn.sigmoid(objectness)
    top_scores, top_idx = jax.lax.top_k(scores, PRE_N)

    pad = NPAD - PRE_N
    pad_anc = jnp.tile(
        jnp.array([[1.0e4, 1.0e4, 0.0, 1.0, 1.0, 1.0, 0.0]], f32), (pad, 1))
    anc_all = jnp.concatenate([anchors_bbox3d, pad_anc], axis=0)
    dl_all = jnp.concatenate([box_regression, jnp.zeros((pad, 7), f32)],
                             axis=0)
    packed = jnp.concatenate(
        [anc_all, jnp.zeros((NROWS, 1), f32),
         dl_all, jnp.zeros((NROWS, 1), f32)], axis=1)

    idxT = jnp.concatenate(
        [top_idx, N_ALL + jnp.arange(pad, dtype=top_idx.dtype)]
    ).astype(jnp.int32)
    scoT = jnp.concatenate(
        [top_scores, jnp.full((pad,), -2.0, f32)]).reshape(NPAD, 1)

    prop, masked = pl.pallas_call(
        _nms_body,
        in_specs=[
            pl.BlockSpec(memory_space=pltpu.SMEM),
            pl.BlockSpec(memory_space=pltpu.VMEM),
            pl.BlockSpec(memory_space=pltpu.VMEM),
        ],
        out_shape=[
            jax.ShapeDtypeStruct((NPAD, 8), f32),
            jax.ShapeDtypeStruct((NPAD, 1), f32),
        ],
        scratch_shapes=[
            pltpu.VMEM((NPAD, 16), f32),
            pltpu.VMEM((NPAD, 8), f32),
            pltpu.VMEM((8, NPAD), f32),
            pltpu.VMEM((1, B), f32),
            pltpu.VMEM((1, B), f32),
        ],
    )(idxT, packed, scoT)

    sel_scores, sel_idx = jax.lax.top_k(masked[:, 0], POST_N)
    selT = jnp.pad(sel_idx.astype(jnp.int32), (0, 24))
    out_g = pl.pallas_call(
        _sel_body,
        in_specs=[
            pl.BlockSpec(memory_space=pltpu.SMEM),
            pl.BlockSpec(memory_space=pltpu.VMEM),
        ],
        out_shape=jax.ShapeDtypeStruct((POST_N + 24, 8), f32),
    )(selT, prop)
    return jnp.concatenate(
        [out_g[:POST_N, :7], sel_scores[:, None]], axis=1)
